# Initial kernel scaffold; baseline (speedup 1.0000x reference)
import jax
import jax.numpy as jnp
from jax.experimental import pallas as pl


def kernel(inputs, embeddings):
    raise NotImplementedError("write your pallas kernel here")


# speedup vs baseline: 13.2636x; 13.2636x over previous
import functools

import jax
import jax.numpy as jnp
from jax import lax
from jax.experimental import pallas as pl
from jax.experimental.pallas import tpu as pltpu
from jax.experimental.pallas import tpu_sc as plsc

B = 16384
L = 200
D = 32
NC = 2
NS = 16
NW = NC * NS
SENT_PER_W = B // NW
C = 8
ROWS = C * L
N_CHUNK = SENT_PER_W // C


def _body(idx_hbm, table_hbm, out_hbm, idx_v, rows_v, out_v, emb0_v, cnt_v, sem):
    wid = lax.axis_index("s") * NC + lax.axis_index("c")

    pltpu.sync_copy(table_hbm.at[pl.ds(0, 1)], emb0_v)
    e0a = emb0_v[0, pl.ds(0, 16)]
    e0b = emb0_v[0, pl.ds(16, 16)]

    zeros = jnp.zeros((16,), jnp.float32)
    lane_hi = lax.iota(jnp.int32, 16) >= 8

    def chunk_body(g, carry):
        sent_base = wid * SENT_PER_W + g * C
        pltpu.sync_copy(idx_hbm.at[pl.ds(sent_base * L, ROWS)], idx_v)
        pltpu.async_copy(table_hbm.at[idx_v], rows_v, sem).wait()

        for s in range(C):
            row0 = s * L

            def lbody(l, acc):
                a0, a1 = acc
                r = row0 + l
                a0 = a0 + rows_v[r, pl.ds(0, 16)]
                a1 = a1 + rows_v[r, pl.ds(16, 16)]
                return (a0, a1)

            a0, a1 = lax.fori_loop(0, L, lbody, (zeros, zeros), unroll=8)

            cnt = jnp.zeros((16,), jnp.int32)
            i_one = jnp.full((16,), 1, jnp.int32)
            i_zero = jnp.zeros((16,), jnp.int32)
            for k in range(12):
                iv = idx_v[pl.ds(row0 + 16 * k, 16)]
                cnt = cnt + jnp.where(iv != 0, i_one, i_zero)
            iv = idx_v[pl.ds(row0 + L - 16, 16)]
            cnt = cnt + jnp.where((iv != 0) & lane_hi, i_one, i_zero)
            t = cnt[0]
            for j in range(1, 16):
                t = t + cnt[j]
            n_valid = jnp.full((16,), 1.0, jnp.float32) * t.astype(jnp.float32)
            n_pad = jnp.float32(L) - n_valid
            scale = 1.0 / (n_valid + 1e-8)
            out_v[s, pl.ds(0, 16)] = (a0 - n_pad * e0a) * scale
            out_v[s, pl.ds(16, 16)] = (a1 - n_pad * e0b) * scale

        pltpu.sync_copy(out_v, out_hbm.at[pl.ds(sent_base, C)])
        return carry

    lax.fori_loop(0, N_CHUNK, chunk_body, 0)


@jax.jit
def _run(idx_flat, embeddings):
    mesh = plsc.VectorSubcoreMesh(core_axis_name="c", subcore_axis_name="s")
    return pl.kernel(
        _body,
        out_type=jax.ShapeDtypeStruct((B, D), jnp.float32),
        mesh=mesh,
        compiler_params=pltpu.CompilerParams(use_tc_tiling_on_sc=False),
        scratch_types=[
            pltpu.VMEM((ROWS,), jnp.int32),
            pltpu.VMEM((ROWS, D), jnp.float32),
            pltpu.VMEM((C, D), jnp.float32),
            pltpu.VMEM((1, D), jnp.float32),
            pltpu.VMEM((16,), jnp.int32),
            pltpu.SemaphoreType.DMA,
        ],
    )(idx_flat, embeddings)


def kernel(inputs, embeddings):
    idx_flat = inputs.astype(jnp.int32).reshape(B * L)
    return _run(idx_flat, embeddings)


# speedup vs baseline: 15.8974x; 1.1986x over previous
import functools

import jax
import jax.numpy as jnp
from jax import lax
from jax.experimental import pallas as pl
from jax.experimental.pallas import tpu as pltpu
from jax.experimental.pallas import tpu_sc as plsc

B = 16384
L = 200
D = 32
NC = 2
NS = 16
NW = NC * NS
SENT_PER_W = B // NW
C = 8
ROWS = C * L
N_CHUNK = SENT_PER_W // C


def _body(idx_hbm, table_hbm, out_hbm,
          idx0, idx1, rows0, rows1, out_v, emb0_v, sem0, sem1):
    wid = lax.axis_index("s") * NC + lax.axis_index("c")
    sent0 = wid * SENT_PER_W

    pltpu.sync_copy(table_hbm.at[pl.ds(0, 1)], emb0_v)
    e0a = emb0_v[0, pl.ds(0, 16)]
    e0b = emb0_v[0, pl.ds(16, 16)]

    zeros = jnp.zeros((16,), jnp.float32)
    lane_hi = lax.iota(jnp.int32, 16) >= 8

    def start_fetch(g, idx_v, rows_v, sem):
        sent_base = sent0 + g * C
        pltpu.sync_copy(idx_hbm.at[pl.ds(sent_base * L, ROWS)], idx_v)
        pltpu.async_copy(table_hbm.at[idx_v], rows_v, sem)

    def wait_fetch(idx_v, rows_v, sem):
        pltpu.make_async_copy(table_hbm.at[idx_v], rows_v, sem).wait()

    def compute_chunk(g, idx_v, rows_v):
        sent_base = sent0 + g * C
        for s in range(C):
            row0 = s * L

            def lbody(l, acc):
                a0, a1 = acc
                r = row0 + l
                a0 = a0 + rows_v[r, pl.ds(0, 16)]
                a1 = a1 + rows_v[r, pl.ds(16, 16)]
                return (a0, a1)

            a0, a1 = lax.fori_loop(0, L, lbody, (zeros, zeros), unroll=8)

            cnt = jnp.zeros((16,), jnp.int32)
            i_one = jnp.full((16,), 1, jnp.int32)
            i_zero = jnp.zeros((16,), jnp.int32)
            for k in range(12):
                iv = idx_v[pl.ds(row0 + 16 * k, 16)]
                cnt = cnt + jnp.where(iv != 0, i_one, i_zero)
            iv = idx_v[pl.ds(row0 + L - 16, 16)]
            cnt = cnt + jnp.where((iv != 0) & lane_hi, i_one, i_zero)
            t = cnt[0]
            for j in range(1, 16):
                t = t + cnt[j]
            n_valid = jnp.full((16,), 1.0, jnp.float32) * t.astype(jnp.float32)
            n_pad = jnp.float32(L) - n_valid
            scale = 1.0 / (n_valid + 1e-8)
            out_v[s, pl.ds(0, 16)] = (a0 - n_pad * e0a) * scale
            out_v[s, pl.ds(16, 16)] = (a1 - n_pad * e0b) * scale

        pltpu.sync_copy(out_v, out_hbm.at[pl.ds(sent_base, C)])

    start_fetch(0, idx0, rows0, sem0)

    def pair_body(g2, carry):
        c0 = 2 * g2
        start_fetch(c0 + 1, idx1, rows1, sem1)
        wait_fetch(idx0, rows0, sem0)
        compute_chunk(c0, idx0, rows0)

        @pl.when(c0 + 2 < N_CHUNK)
        def _():
            start_fetch(c0 + 2, idx0, rows0, sem0)

        wait_fetch(idx1, rows1, sem1)
        compute_chunk(c0 + 1, idx1, rows1)
        return carry

    lax.fori_loop(0, N_CHUNK // 2, pair_body, 0)


@jax.jit
def _run(idx_flat, embeddings):
    mesh = plsc.VectorSubcoreMesh(core_axis_name="c", subcore_axis_name="s")
    return pl.kernel(
        _body,
        out_type=jax.ShapeDtypeStruct((B, D), jnp.float32),
        mesh=mesh,
        compiler_params=pltpu.CompilerParams(use_tc_tiling_on_sc=False),
        scratch_types=[
            pltpu.VMEM((ROWS,), jnp.int32),
            pltpu.VMEM((ROWS,), jnp.int32),
            pltpu.VMEM((ROWS, D), jnp.float32),
            pltpu.VMEM((ROWS, D), jnp.float32),
            pltpu.VMEM((C, D), jnp.float32),
            pltpu.VMEM((1, D), jnp.float32),
            pltpu.SemaphoreType.DMA,
            pltpu.SemaphoreType.DMA,
        ],
    )(idx_flat, embeddings)


def kernel(inputs, embeddings):
    idx_flat = inputs.astype(jnp.int32).reshape(B * L)
    return _run(idx_flat, embeddings)
